# baseline (speedup 1.0000x reference)
import functools

import jax
import jax.numpy as jnp
from jax import lax
from jax.experimental import pallas as pl
from jax.experimental.pallas import tpu as pltpu
from jax.experimental.pallas import tpu_sc as plsc

N = 10000
E = 320000
C = 128

NC = 2
NS = 16
NW = NC * NS
EPW = E // NW
SUB = 80
NPAD = 10240
NPT = NPAD // NS
BE = 1280

PART_CHUNKS = (63, 62)
PART_SIZES = tuple(a * SUB * NW for a in PART_CHUNKS)
assert sum(PART_SIZES) == E and all(p % BE == 0 for p in PART_SIZES)

_f32 = jnp.float32
_i32 = jnp.int32


def _wid():
    return lax.axis_index("c") * NS + lax.axis_index("s")


def _pingpong(nch, fire, drain):
    fire(0, 0)
    if nch % 2 == 1:
        h_iters = (nch - 1) // 2
    else:
        h_iters = (nch - 2) // 2

    def body(h, carry):
        g = h * 2
        fire(g + 1, 1)
        drain(g, 0)
        fire(g + 2, 0)
        drain(g + 1, 1)
        return carry

    lax.fori_loop(0, h_iters, body, 0)
    if nch % 2 == 1:
        drain(nch - 1, 0)
    else:
        g = nch - 2
        fire(g + 1, 1)
        drain(g, 0)
        drain(g + 1, 1)


def _tables_tc(nf, w1i, w1j):
    bn = 1000

    def body(nf_ref, wi_ref, wj_ref, ts_ref, tr_ref):
        nfb = nf_ref[...]
        ts_ref[...] = jnp.dot(nfb, wi_ref[...], preferred_element_type=_f32)
        tr_ref[...] = jnp.dot(nfb, wj_ref[...], preferred_element_type=_f32)

    return pl.pallas_call(
        body,
        grid=(N // bn,),
        in_specs=[
            pl.BlockSpec((bn, C), lambda i: (i, 0)),
            pl.BlockSpec((C, C), lambda i: (0, 0)),
            pl.BlockSpec((C, C), lambda i: (0, 0)),
        ],
        out_specs=[pl.BlockSpec((bn, C), lambda i: (i, 0))] * 2,
        out_shape=[jax.ShapeDtypeStruct((N, C), _f32)] * 2,
    )(nf, w1i, w1j)


def _sc_geo(cx_a, cy_a, cz_a, snd, rcv):
    mesh = plsc.VectorSubcoreMesh(core_axis_name="c", subcore_axis_name="s")

    @functools.partial(
        pl.kernel,
        out_type=jax.ShapeDtypeStruct((E * 4,), _f32),
        mesh=mesh,
        scratch_types=(
            pltpu.VMEM((N,), _f32),
            pltpu.VMEM((N,), _f32),
            pltpu.VMEM((N,), _f32),
            pltpu.VMEM((EPW,), _i32),
            pltpu.VMEM((EPW,), _i32),
            pltpu.VMEM((EPW * 4,), _f32),
        ),
        compiler_params=pltpu.CompilerParams(needs_layout_passes=False),
    )
    def k(cx_h, cy_h, cz_h, snd_h, rcv_h, rv_h, cx, cy, cz, ixs, ixr, rbuf):
        base = pl.multiple_of(_wid() * EPW, 8)
        pltpu.sync_copy(cx_h, cx)
        pltpu.sync_copy(cy_h, cy)
        pltpu.sync_copy(cz_h, cz)
        pltpu.sync_copy(snd_h.at[pl.ds(base, EPW)], ixs)
        pltpu.sync_copy(rcv_h.at[pl.ds(base, EPW)], ixr)
        lane = lax.iota(_i32, 16)

        def body(g, carry):
            o16 = pl.multiple_of(g * 16, 8)
            s16 = ixs[pl.ds(o16, 16)]
            r16 = ixr[pl.ds(o16, 16)]
            flat = (g * 64) + lane * 4
            for comp, cref in ((0, cx), (1, cy), (2, cz)):
                ci = plsc.load_gather(cref, [s16])
                cj = plsc.load_gather(cref, [r16])
                plsc.store_scatter(rbuf, [flat + comp], ci - cj)
            return carry

        lax.fori_loop(0, EPW // 16, body, 0)
        pltpu.sync_copy(rbuf, rv_h.at[pl.ds(base * 4, EPW * 4)])

    return k(cx_a, cy_a, cz_a, snd, rcv)


def _sc_gather(ts, tr, snd_p, rcv_p, ep):
    epw = ep // NW
    nch = epw // SUB
    mesh = plsc.VectorSubcoreMesh(core_axis_name="c", subcore_axis_name="s")

    @functools.partial(
        pl.kernel,
        out_type=(
            jax.ShapeDtypeStruct((ep, C), _f32),
            jax.ShapeDtypeStruct((ep, C), _f32),
        ),
        mesh=mesh,
        scratch_types=(
            pltpu.VMEM((epw,), _i32),
            pltpu.VMEM((epw,), _i32),
            pltpu.VMEM((SUB, C), _f32),
            pltpu.VMEM((SUB, C), _f32),
            pltpu.VMEM((SUB, C), _f32),
            pltpu.VMEM((SUB, C), _f32),
            pltpu.SemaphoreType.DMA,
            pltpu.SemaphoreType.DMA,
        ),
    )
    def k(ts_h, tr_h, snd_h, rcv_h, ga_h, gb_h, ixs, ixr, rs0, rr0, rs1, rr1,
          sem0, sem1):
        base = pl.multiple_of(_wid() * epw, 8)
        pltpu.sync_copy(snd_h.at[pl.ds(base, epw)], ixs)
        pltpu.sync_copy(rcv_h.at[pl.ds(base, epw)], ixr)
        slots = ((rs0, rr0, sem0), (rs1, rr1, sem1))

        def fire(g, slot):
            rs, rr, sem = slots[slot]
            isl = pl.ds(pl.multiple_of(g * SUB, 8), SUB)
            pltpu.async_copy(ts_h.at[ixs.at[isl]], rs, sem)
            pltpu.async_copy(tr_h.at[ixr.at[isl]], rr, sem)

        def drain(g, slot):
            rs, rr, sem = slots[slot]
            isl = pl.ds(pl.multiple_of(g * SUB, 8), SUB)
            pltpu.make_async_copy(ts_h.at[ixs.at[isl]], rs, sem).wait()
            pltpu.make_async_copy(tr_h.at[ixr.at[isl]], rr, sem).wait()
            off = pl.multiple_of(base + g * SUB, 8)
            pltpu.sync_copy(rs, ga_h.at[pl.ds(off, SUB)])
            pltpu.sync_copy(rr, gb_h.at[pl.ds(off, SUB)])

        _pingpong(nch, fire, drain)

    return k(ts, tr, snd_p, rcv_p)


_SINCOEF = (6.28318531, -41.34170217, 81.60524536, -76.70576095,
            42.05737007, -15.08455476, 3.77595755, -0.61505996)


def _sin2pi(f):
    f = f - jnp.round(f)
    x2 = f * f
    p = jnp.float32(_SINCOEF[-1])
    for coef in _SINCOEF[-2::-1]:
        p = p * x2 + jnp.float32(coef)
    return f * p


def _edge_tc(ga, gb, rv, zc16, w1g16, w_abs, b1, w2, b2, wax1, bax1, wax2,
             bax2, ep):
    def body(ga_ref, gb_ref, rv_ref, zc_ref, w1g_ref, wab_ref, b1_ref, w2_ref,
             b2_ref, wax1_ref, bax1_ref, wax2_ref, bax2_ref,
             msg_ref, dlt_ref, att_ref):
        r = rv_ref[:, :3]
        a = jnp.sqrt(jnp.sum(r * r, axis=1, keepdims=True))
        rbf16 = _sin2pi(a * zc_ref[...]) / a
        geo = jnp.dot(rbf16, w1g_ref[...], preferred_element_type=_f32)
        geo = geo + a * wab_ref[...]
        h1 = jax.nn.silu(ga_ref[...] + gb_ref[...] + geo + b1_ref[...])
        m = jnp.dot(h1, w2_ref[...], preferred_element_type=_f32) + b2_ref[...]
        hax = jax.nn.silu(jnp.dot(m, wax1_ref[...], preferred_element_type=_f32)
                          + bax1_ref[...])
        out2 = jnp.dot(hax, wax2_ref[...], preferred_element_type=_f32) \
            + bax2_ref[...]
        att = jax.nn.sigmoid(out2[:, 0:1])
        px = out2[:, 1:2]
        delta = r * (px / (a + 1.0))
        msg_ref[...] = m * att
        dlt_ref[...] = jnp.concatenate([delta, jnp.zeros((BE, 1), _f32)], axis=1)
        att_ref[...] = att

    full = lambda shape: pl.BlockSpec(shape, lambda i: (0, 0))
    return pl.pallas_call(
        body,
        grid=(ep // BE,),
        in_specs=[
            pl.BlockSpec((BE, C), lambda i: (i, 0)),
            pl.BlockSpec((BE, C), lambda i: (i, 0)),
            pl.BlockSpec((BE, 4), lambda i: (i, 0)),
            full((1, 16)),
            full((16, C)),
            full((1, C)),
            full((1, C)),
            full((C, C)),
            full((1, C)),
            full((C, 2 * C)),
            full((1, 2 * C)),
            full((2 * C, C)),
            full((1, C)),
        ],
        out_specs=[
            pl.BlockSpec((BE, C), lambda i: (i, 0)),
            pl.BlockSpec((BE, 4), lambda i: (i, 0)),
            pl.BlockSpec((BE, 1), lambda i: (i, 0)),
        ],
        out_shape=[
            jax.ShapeDtypeStruct((ep, C), _f32),
            jax.ShapeDtypeStruct((ep, 4), _f32),
            jax.ShapeDtypeStruct((ep, 1), _f32),
        ],
    )(ga, gb, rv, zc16, w1g16, w_abs, b1, w2, b2, wax1, bax1, wax2, bax2)


def _sc_scatter_msg(msg_p, snd_p, zrows, ep):
    epw = ep // NW
    nch = epw // SUB
    mesh = plsc.VectorSubcoreMesh(core_axis_name="c", subcore_axis_name="s")

    @functools.partial(
        pl.kernel,
        out_type=jax.ShapeDtypeStruct((NC, NPAD, C), _f32),
        mesh=mesh,
        scratch_types=(
            pltpu.VMEM((SUB,), _i32),
            pltpu.VMEM((SUB,), _i32),
            pltpu.VMEM((SUB, C), _f32),
            pltpu.VMEM((SUB, C), _f32),
            pltpu.VMEM_SHARED((NPAD, C), _f32),
            pltpu.SemaphoreType.DMA,
            pltpu.SemaphoreType.DMA,
        ),
    )
    def k(msg_h, snd_h, z_h, outm_h, ix0, ix1, rows0, rows1, acc, sem0, sem1):
        c = lax.axis_index("c")
        s = lax.axis_index("s")
        roff = pl.multiple_of(s * NPT, 8)
        pltpu.sync_copy(z_h.at[pl.ds(roff, NPT)], acc.at[pl.ds(roff, NPT)])
        plsc.subcore_barrier()
        base = pl.multiple_of((c * NS + s) * epw, 8)
        slots = ((ix0, rows0, sem0), (ix1, rows1, sem1))

        def fire(g, slot):
            ix, rows, sem = slots[slot]
            off = pl.multiple_of(base + g * SUB, 8)
            pltpu.async_copy(snd_h.at[pl.ds(off, SUB)], ix, sem)
            pltpu.async_copy(msg_h.at[pl.ds(off, SUB)], rows, sem)

        def drain(g, slot):
            ix, rows, sem = slots[slot]
            off = pl.multiple_of(base + g * SUB, 8)
            pltpu.make_async_copy(snd_h.at[pl.ds(off, SUB)], ix, sem).wait()
            pltpu.make_async_copy(msg_h.at[pl.ds(off, SUB)], rows, sem).wait()
            pltpu.sync_copy(rows, acc.at[ix], add=True)

        _pingpong(nch, fire, drain)
        plsc.subcore_barrier()
        pltpu.sync_copy(acc.at[pl.ds(roff, NPT)], outm_h.at[c, pl.ds(roff, NPT)])

    return k(msg_p, snd_p, zrows)


def _sc_scatter_delta(dvec_p, snd_p, ep):
    epw = ep // NW
    nch = epw // SUB
    mesh = plsc.VectorSubcoreMesh(core_axis_name="c", subcore_axis_name="s")

    @functools.partial(
        pl.kernel,
        out_type=jax.ShapeDtypeStruct((NW * N * 4,), _f32),
        mesh=mesh,
        scratch_types=(
            pltpu.VMEM((SUB,), _i32),
            pltpu.VMEM((SUB,), _i32),
            pltpu.VMEM((SUB * 4,), _f32),
            pltpu.VMEM((SUB * 4,), _f32),
            pltpu.VMEM((N * 4,), _f32),
            pltpu.SemaphoreType.DMA,
            pltpu.SemaphoreType.DMA,
        ),
        compiler_params=pltpu.CompilerParams(needs_layout_passes=False),
    )
    def k(dv_h, snd_h, outd_h, ix0, ix1, dbuf0, dbuf1, dacc, sem0, sem1):
        wid = _wid()
        z16 = jnp.zeros((16,), _f32)

        def zbody(g, carry):
            dacc[pl.ds(pl.multiple_of(g * 16, 8), 16)] = z16
            return carry

        lax.fori_loop(0, N * 4 // 16, zbody, 0)
        base = pl.multiple_of(wid * epw, 8)
        lane = lax.iota(_i32, 16)
        slots = ((ix0, dbuf0, sem0), (ix1, dbuf1, sem1))

        def fire(g, slot):
            ix, dbuf, sem = slots[slot]
            off = pl.multiple_of(base + g * SUB, 8)
            pltpu.async_copy(snd_h.at[pl.ds(off, SUB)], ix, sem)
            pltpu.async_copy(dv_h.at[pl.ds(off * 4, SUB * 4)], dbuf, sem)

        def drain(g, slot):
            ix, dbuf, sem = slots[slot]
            off = pl.multiple_of(base + g * SUB, 8)
            pltpu.make_async_copy(snd_h.at[pl.ds(off, SUB)], ix, sem).wait()
            pltpu.make_async_copy(dv_h.at[pl.ds(off * 4, SUB * 4)], dbuf,
                                  sem).wait()
            for q in range(SUB // 16):
                s16 = ix[pl.ds(q * 16, 16)]
                src = q * 64 + lane * 4
                for comp in range(3):
                    vals = plsc.load_gather(dbuf, [src + comp])
                    plsc.addupdate_scatter(dacc, [s16 * 4 + comp], vals)

        _pingpong(nch, fire, drain)
        pltpu.sync_copy(dacc, outd_h.at[pl.ds(pl.multiple_of(wid * N * 4, 8),
                                              N * 4)])

    return k(dvec_p, snd_p)


def _delta_combine(dparts2d):
    ndp = dparts2d.shape[0]

    def body(dp_ref, out_ref):
        out_ref[...] = jnp.sum(dp_ref[...], axis=0, keepdims=True)

    return pl.pallas_call(
        body,
        grid=(1,),
        in_specs=[pl.BlockSpec((ndp, N * 4), lambda i: (0, 0))],
        out_specs=pl.BlockSpec((1, N * 4), lambda i: (0, 0)),
        out_shape=jax.ShapeDtypeStruct((1, N * 4), _f32),
    )(dparts2d)


def _node_tc(nf, coords, msg_parts, dsum, wn1a, wn1b, bn1, wn2, bn2):
    bn = 1000
    nparts = len(msg_parts)

    def body(nf_ref, co_ref, *refs):
        p_refs = refs[:nparts]
        dp_ref = refs[nparts]
        wa_ref, wb_ref, b1_ref, w2_ref, b2_ref, nfo_ref, coo_ref = refs[nparts + 1:]
        m = p_refs[0][...]
        for p in p_refs[1:]:
            m = m + p[...]
        delta = dp_ref[:, :3]
        nfb = nf_ref[...]
        h = jax.nn.silu(jnp.dot(nfb, wa_ref[...], preferred_element_type=_f32)
                        + jnp.dot(m, wb_ref[...], preferred_element_type=_f32)
                        + b1_ref[...])
        nfo_ref[...] = jnp.dot(h, w2_ref[...], preferred_element_type=_f32) \
            + b2_ref[...] + nfb
        coo_ref[...] = co_ref[...] + delta

    full = lambda shape: pl.BlockSpec(shape, lambda i: (0, 0))
    return pl.pallas_call(
        body,
        grid=(N // bn,),
        in_specs=[
            pl.BlockSpec((bn, C), lambda i: (i, 0)),
            pl.BlockSpec((bn, 3), lambda i: (i, 0)),
        ] + [pl.BlockSpec((bn, C), lambda i: (i, 0))] * nparts + [
            pl.BlockSpec((bn, 4), lambda i: (i, 0)),
            full((C, C)),
            full((C, C)),
            full((1, C)),
            full((C, C)),
            full((1, C)),
        ],
        out_specs=[
            pl.BlockSpec((bn, C), lambda i: (i, 0)),
            pl.BlockSpec((bn, 3), lambda i: (i, 0)),
        ],
        out_shape=[
            jax.ShapeDtypeStruct((N, C), _f32),
            jax.ShapeDtypeStruct((N, 3), _f32),
        ],
    )(nf, coords, *msg_parts, dsum, wn1a, wn1b, bn1, wn2, bn2)


def kernel(node_feats, coordinates, edge_index, params):
    pe, pn, pa, px = params["phi_e"], params["phi_n"], params["att"], params["phi_x"]
    w1 = pe["W1"]
    w1i = w1[:C]
    w1j = w1[C:2 * C]
    w_abs = w1[2 * C:2 * C + 1]
    cut = params["bessel_cut_off"]
    amp = jnp.sqrt(2.0 / cut)
    zc16 = jnp.zeros((1, 16), _f32).at[0, :8].set(
        params["z_0k"] / (2.0 * jnp.pi * cut))
    w1g16 = jnp.zeros((16, C), _f32).at[:8].set(w1[2 * C + 1:] * amp)
    b1 = pe["b1"].reshape(1, C)
    b2 = pe["b2"].reshape(1, C)
    wax1 = jnp.concatenate([pa["W1"], px["W1"]], axis=1)
    bax1 = jnp.concatenate([pa["b1"], px["b1"]]).reshape(1, 2 * C)
    wax2 = jnp.zeros((2 * C, C), _f32)
    wax2 = wax2.at[:C, 0].set(pa["W2"][:, 0]).at[C:, 1].set(px["W2"][:, 0])
    bax2 = jnp.zeros((1, C), _f32)
    bax2 = bax2.at[0, 0].set(pa["b2"][0]).at[0, 1].set(px["b2"][0])
    wn1 = pn["W1"]
    wn1a, wn1b = wn1[:C], wn1[C:]
    bn1 = pn["b1"].reshape(1, C)
    bn2 = pn["b2"].reshape(1, C)

    snd = edge_index[0]
    rcv = edge_index[1]
    cxyz = coordinates.T
    zrows = jnp.zeros((NPAD, C), _f32)

    ts, tr = _tables_tc(node_feats, w1i, w1j)
    rv = _sc_geo(cxyz[0], cxyz[1], cxyz[2], snd, rcv).reshape(E, 4)

    msg_parts, dparts_list, att_parts = [], [], []
    off = 0
    for ep in PART_SIZES:
        snd_p = lax.slice(snd, (off,), (off + ep,))
        rcv_p = lax.slice(rcv, (off,), (off + ep,))
        rv_p = lax.slice(rv, (off, 0), (off + ep, 4))
        ga, gb = _sc_gather(ts, tr, snd_p, rcv_p, ep)
        msg, dvec, att = _edge_tc(ga, gb, rv_p, zc16, w1g16, w_abs, b1,
                                  pe["W2"], b2, wax1, bax1, wax2, bax2, ep)
        partm = _sc_scatter_msg(msg, snd_p, zrows, ep)
        partd = _sc_scatter_delta(dvec.reshape(ep * 4), snd_p, ep)
        msg_parts.extend([partm[0], partm[1]])
        dparts_list.append(partd)
        att_parts.append(att)
        off += ep

    dparts2d = jnp.concatenate(dparts_list).reshape(len(PART_SIZES) * NW, N * 4)
    dsum = _delta_combine(dparts2d).reshape(N, 4)
    nf_new, co_new = _node_tc(node_feats, coordinates, msg_parts, dsum,
                              wn1a, wn1b, bn1, pn["W2"], bn2)
    att_full = jnp.concatenate(att_parts, axis=0)
    return nf_new, co_new, att_full


# speedup vs baseline: 1.2002x; 1.2002x over previous
import functools

import jax
import jax.numpy as jnp
from jax import lax
from jax.experimental import pallas as pl
from jax.experimental.pallas import tpu as pltpu
from jax.experimental.pallas import tpu_sc as plsc

N = 10000
E = 320000
C = 128

NC = 2
NS = 16
NW = NC * NS
EPW = E // NW
SUB = 80
NPAD = 10240
NPT = NPAD // NS
BE = 2560

PART_CHUNKS = (125,)
PART_SIZES = tuple(a * SUB * NW for a in PART_CHUNKS)
assert sum(PART_SIZES) == E and all(p % BE == 0 for p in PART_SIZES)

_f32 = jnp.float32
_i32 = jnp.int32


def _wid():
    return lax.axis_index("c") * NS + lax.axis_index("s")


def _pingpong(nch, fire, drain):
    fire(0, 0)
    if nch % 2 == 1:
        h_iters = (nch - 1) // 2
    else:
        h_iters = (nch - 2) // 2

    def body(h, carry):
        g = h * 2
        fire(g + 1, 1)
        drain(g, 0)
        fire(g + 2, 0)
        drain(g + 1, 1)
        return carry

    lax.fori_loop(0, h_iters, body, 0)
    if nch % 2 == 1:
        drain(nch - 1, 0)
    else:
        g = nch - 2
        fire(g + 1, 1)
        drain(g, 0)
        drain(g + 1, 1)


def _tables_tc(nf, w1i, w1j):
    bn = 1000

    def body(nf_ref, wi_ref, wj_ref, ts_ref, tr_ref):
        nfb = nf_ref[...]
        ts_ref[...] = jnp.dot(nfb, wi_ref[...], preferred_element_type=_f32)
        tr_ref[...] = jnp.dot(nfb, wj_ref[...], preferred_element_type=_f32)

    return pl.pallas_call(
        body,
        grid=(N // bn,),
        in_specs=[
            pl.BlockSpec((bn, C), lambda i: (i, 0)),
            pl.BlockSpec((C, C), lambda i: (0, 0)),
            pl.BlockSpec((C, C), lambda i: (0, 0)),
        ],
        out_specs=[pl.BlockSpec((bn, C), lambda i: (i, 0))] * 2,
        out_shape=[jax.ShapeDtypeStruct((N, C), _f32)] * 2,
    )(nf, w1i, w1j)


def _sc_geo(cx_a, cy_a, cz_a, snd, rcv):
    mesh = plsc.VectorSubcoreMesh(core_axis_name="c", subcore_axis_name="s")

    @functools.partial(
        pl.kernel,
        out_type=jax.ShapeDtypeStruct((E * 4,), _f32),
        mesh=mesh,
        scratch_types=(
            pltpu.VMEM((N,), _f32),
            pltpu.VMEM((N,), _f32),
            pltpu.VMEM((N,), _f32),
            pltpu.VMEM((EPW,), _i32),
            pltpu.VMEM((EPW,), _i32),
            pltpu.VMEM((EPW * 4,), _f32),
        ),
        compiler_params=pltpu.CompilerParams(needs_layout_passes=False),
    )
    def k(cx_h, cy_h, cz_h, snd_h, rcv_h, rv_h, cx, cy, cz, ixs, ixr, rbuf):
        base = pl.multiple_of(_wid() * EPW, 8)
        pltpu.sync_copy(cx_h, cx)
        pltpu.sync_copy(cy_h, cy)
        pltpu.sync_copy(cz_h, cz)
        pltpu.sync_copy(snd_h.at[pl.ds(base, EPW)], ixs)
        pltpu.sync_copy(rcv_h.at[pl.ds(base, EPW)], ixr)
        lane = lax.iota(_i32, 16)

        def body(g, carry):
            o16 = pl.multiple_of(g * 16, 8)
            s16 = ixs[pl.ds(o16, 16)]
            r16 = ixr[pl.ds(o16, 16)]
            flat = (g * 64) + lane * 4
            for comp, cref in ((0, cx), (1, cy), (2, cz)):
                ci = plsc.load_gather(cref, [s16])
                cj = plsc.load_gather(cref, [r16])
                plsc.store_scatter(rbuf, [flat + comp], ci - cj)
            return carry

        lax.fori_loop(0, EPW // 16, body, 0)
        pltpu.sync_copy(rbuf, rv_h.at[pl.ds(base * 4, EPW * 4)])

    return k(cx_a, cy_a, cz_a, snd, rcv)


def _sc_gather(ts, tr, snd_p, rcv_p, ep):
    epw = ep // NW
    nch = epw // SUB
    mesh = plsc.VectorSubcoreMesh(core_axis_name="c", subcore_axis_name="s")

    @functools.partial(
        pl.kernel,
        out_type=(
            jax.ShapeDtypeStruct((ep, C), _f32),
            jax.ShapeDtypeStruct((ep, C), _f32),
        ),
        mesh=mesh,
        scratch_types=(
            pltpu.VMEM((epw,), _i32),
            pltpu.VMEM((epw,), _i32),
            pltpu.VMEM((SUB, C), _f32),
            pltpu.VMEM((SUB, C), _f32),
            pltpu.VMEM((SUB, C), _f32),
            pltpu.VMEM((SUB, C), _f32),
            pltpu.SemaphoreType.DMA,
            pltpu.SemaphoreType.DMA,
        ),
    )
    def k(ts_h, tr_h, snd_h, rcv_h, ga_h, gb_h, ixs, ixr, rs0, rr0, rs1, rr1,
          sem0, sem1):
        base = pl.multiple_of(_wid() * epw, 8)
        pltpu.sync_copy(snd_h.at[pl.ds(base, epw)], ixs)
        pltpu.sync_copy(rcv_h.at[pl.ds(base, epw)], ixr)
        slots = ((rs0, rr0, sem0), (rs1, rr1, sem1))

        def fire(g, slot):
            rs, rr, sem = slots[slot]
            isl = pl.ds(pl.multiple_of(g * SUB, 8), SUB)
            pltpu.async_copy(ts_h.at[ixs.at[isl]], rs, sem)
            pltpu.async_copy(tr_h.at[ixr.at[isl]], rr, sem)

        def drain(g, slot):
            rs, rr, sem = slots[slot]
            isl = pl.ds(pl.multiple_of(g * SUB, 8), SUB)
            pltpu.make_async_copy(ts_h.at[ixs.at[isl]], rs, sem).wait()
            pltpu.make_async_copy(tr_h.at[ixr.at[isl]], rr, sem).wait()
            off = pl.multiple_of(base + g * SUB, 8)
            pltpu.sync_copy(rs, ga_h.at[pl.ds(off, SUB)])
            pltpu.sync_copy(rr, gb_h.at[pl.ds(off, SUB)])

        _pingpong(nch, fire, drain)

    return k(ts, tr, snd_p, rcv_p)


_SINCOEF = (6.28318531, -41.34170217, 81.60524536, -76.70576095,
            42.05737007, -15.08455476, 3.77595755, -0.61505996)


def _sin2pi(f):
    f = f - jnp.round(f)
    x2 = f * f
    p = jnp.float32(_SINCOEF[-1])
    for coef in _SINCOEF[-2::-1]:
        p = p * x2 + jnp.float32(coef)
    return f * p


def _edge_tc(ga, gb, rv, zc16, w1g16, w_abs, b1, w2, b2, wax1, bax1, wax2,
             bax2, ep):
    def body(ga_ref, gb_ref, rv_ref, zc_ref, w1g_ref, wab_ref, b1_ref, w2_ref,
             b2_ref, wax1_ref, bax1_ref, wax2_ref, bax2_ref,
             msg_ref, dlt_ref, att_ref):
        r = rv_ref[:, :3]
        a = jnp.sqrt(jnp.sum(r * r, axis=1, keepdims=True))
        rbf16 = _sin2pi(a * zc_ref[...]) / a
        rbf16 = rbf16 + a * wab_ref[...] + b1_ref[...]
        geo = jnp.dot(rbf16, w1g_ref[...], preferred_element_type=_f32)
        h1 = jax.nn.silu(ga_ref[...] + gb_ref[...] + geo)
        m = jnp.dot(h1, w2_ref[...], preferred_element_type=_f32) + b2_ref[...]
        hax = jax.nn.silu(jnp.dot(m, wax1_ref[...], preferred_element_type=_f32)
                          + bax1_ref[...])
        out2 = jnp.dot(hax, wax2_ref[...], preferred_element_type=_f32) \
            + bax2_ref[...]
        att = jax.nn.sigmoid(out2[:, 0:1])
        px = out2[:, 1:2]
        msg_ref[...] = m * att
        dlt_ref[...] = rv_ref[...] * (px / (a + 1.0))
        att_ref[...] = att

    full = lambda shape: pl.BlockSpec(shape, lambda i: (0, 0))
    return pl.pallas_call(
        body,
        grid=(ep // BE,),
        in_specs=[
            pl.BlockSpec((BE, C), lambda i: (i, 0)),
            pl.BlockSpec((BE, C), lambda i: (i, 0)),
            pl.BlockSpec((BE, 4), lambda i: (i, 0)),
            full((1, 16)),
            full((16, C)),
            full((1, 16)),
            full((1, 16)),
            full((C, C)),
            full((1, C)),
            full((C, 2 * C)),
            full((1, 2 * C)),
            full((2 * C, C)),
            full((1, C)),
        ],
        out_specs=[
            pl.BlockSpec((BE, C), lambda i: (i, 0)),
            pl.BlockSpec((BE, 4), lambda i: (i, 0)),
            pl.BlockSpec((BE, 1), lambda i: (i, 0)),
        ],
        out_shape=[
            jax.ShapeDtypeStruct((ep, C), _f32),
            jax.ShapeDtypeStruct((ep, 4), _f32),
            jax.ShapeDtypeStruct((ep, 1), _f32),
        ],
    )(ga, gb, rv, zc16, w1g16, w_abs, b1, w2, b2, wax1, bax1, wax2, bax2)


def _sc_scatter_msg(msg_p, snd_p, zrows, ep):
    epw = ep // NW
    nch = epw // SUB
    mesh = plsc.VectorSubcoreMesh(core_axis_name="c", subcore_axis_name="s")

    @functools.partial(
        pl.kernel,
        out_type=jax.ShapeDtypeStruct((NC, NPAD, C), _f32),
        mesh=mesh,
        scratch_types=(
            pltpu.VMEM((SUB,), _i32),
            pltpu.VMEM((SUB,), _i32),
            pltpu.VMEM((SUB, C), _f32),
            pltpu.VMEM((SUB, C), _f32),
            pltpu.VMEM_SHARED((NPAD, C), _f32),
            pltpu.SemaphoreType.DMA,
            pltpu.SemaphoreType.DMA,
        ),
    )
    def k(msg_h, snd_h, z_h, outm_h, ix0, ix1, rows0, rows1, acc, sem0, sem1):
        c = lax.axis_index("c")
        s = lax.axis_index("s")
        roff = pl.multiple_of(s * NPT, 8)
        pltpu.sync_copy(z_h.at[pl.ds(roff, NPT)], acc.at[pl.ds(roff, NPT)])
        plsc.subcore_barrier()
        base = pl.multiple_of((c * NS + s) * epw, 8)
        slots = ((ix0, rows0, sem0), (ix1, rows1, sem1))

        def fire(g, slot):
            ix, rows, sem = slots[slot]
            off = pl.multiple_of(base + g * SUB, 8)
            pltpu.async_copy(snd_h.at[pl.ds(off, SUB)], ix, sem)
            pltpu.async_copy(msg_h.at[pl.ds(off, SUB)], rows, sem)

        def drain(g, slot):
            ix, rows, sem = slots[slot]
            off = pl.multiple_of(base + g * SUB, 8)
            pltpu.make_async_copy(snd_h.at[pl.ds(off, SUB)], ix, sem).wait()
            pltpu.make_async_copy(msg_h.at[pl.ds(off, SUB)], rows, sem).wait()
            pltpu.sync_copy(rows, acc.at[ix], add=True)

        _pingpong(nch, fire, drain)
        plsc.subcore_barrier()
        pltpu.sync_copy(acc.at[pl.ds(roff, NPT)], outm_h.at[c, pl.ds(roff, NPT)])

    return k(msg_p, snd_p, zrows)


def _sc_scatter_delta(dvec_p, snd_p, ep):
    epw = ep // NW
    nch = epw // SUB
    mesh = plsc.VectorSubcoreMesh(core_axis_name="c", subcore_axis_name="s")

    @functools.partial(
        pl.kernel,
        out_type=jax.ShapeDtypeStruct((NW * N * 4,), _f32),
        mesh=mesh,
        scratch_types=(
            pltpu.VMEM((SUB,), _i32),
            pltpu.VMEM((SUB,), _i32),
            pltpu.VMEM((SUB * 4,), _f32),
            pltpu.VMEM((SUB * 4,), _f32),
            pltpu.VMEM((N * 4,), _f32),
            pltpu.SemaphoreType.DMA,
            pltpu.SemaphoreType.DMA,
        ),
        compiler_params=pltpu.CompilerParams(needs_layout_passes=False),
    )
    def k(dv_h, snd_h, outd_h, ix0, ix1, dbuf0, dbuf1, dacc, sem0, sem1):
        wid = _wid()
        z16 = jnp.zeros((16,), _f32)

        def zbody(g, carry):
            dacc[pl.ds(pl.multiple_of(g * 16, 8), 16)] = z16
            return carry

        lax.fori_loop(0, N * 4 // 16, zbody, 0)
        base = pl.multiple_of(wid * epw, 8)
        lane = lax.iota(_i32, 16)
        slots = ((ix0, dbuf0, sem0), (ix1, dbuf1, sem1))

        def fire(g, slot):
            ix, dbuf, sem = slots[slot]
            off = pl.multiple_of(base + g * SUB, 8)
            pltpu.async_copy(snd_h.at[pl.ds(off, SUB)], ix, sem)
            pltpu.async_copy(dv_h.at[pl.ds(off * 4, SUB * 4)], dbuf, sem)

        def drain(g, slot):
            ix, dbuf, sem = slots[slot]
            off = pl.multiple_of(base + g * SUB, 8)
            pltpu.make_async_copy(snd_h.at[pl.ds(off, SUB)], ix, sem).wait()
            pltpu.make_async_copy(dv_h.at[pl.ds(off * 4, SUB * 4)], dbuf,
                                  sem).wait()
            for q in range(SUB // 16):
                s16 = ix[pl.ds(q * 16, 16)]
                src = q * 64 + lane * 4
                for comp in range(3):
                    vals = plsc.load_gather(dbuf, [src + comp])
                    plsc.addupdate_scatter(dacc, [s16 * 4 + comp], vals)

        _pingpong(nch, fire, drain)
        pltpu.sync_copy(dacc, outd_h.at[pl.ds(pl.multiple_of(wid * N * 4, 8),
                                              N * 4)])

    return k(dvec_p, snd_p)


def _delta_combine(dparts2d):
    ndp = dparts2d.shape[0]

    def body(dp_ref, out_ref):
        out_ref[...] = jnp.sum(dp_ref[...], axis=0, keepdims=True)

    return pl.pallas_call(
        body,
        grid=(1,),
        in_specs=[pl.BlockSpec((ndp, N * 4), lambda i: (0, 0))],
        out_specs=pl.BlockSpec((1, N * 4), lambda i: (0, 0)),
        out_shape=jax.ShapeDtypeStruct((1, N * 4), _f32),
    )(dparts2d)


def _node_tc(nf, coords, msg_parts, dsum, wn1a, wn1b, bn1, wn2, bn2):
    bn = 1000
    nparts = len(msg_parts)

    def body(nf_ref, co_ref, *refs):
        p_refs = refs[:nparts]
        dp_ref = refs[nparts]
        wa_ref, wb_ref, b1_ref, w2_ref, b2_ref, nfo_ref, coo_ref = refs[nparts + 1:]
        m = p_refs[0][...]
        for p in p_refs[1:]:
            m = m + p[...]
        delta = dp_ref[:, :3]
        nfb = nf_ref[...]
        h = jax.nn.silu(jnp.dot(nfb, wa_ref[...], preferred_element_type=_f32)
                        + jnp.dot(m, wb_ref[...], preferred_element_type=_f32)
                        + b1_ref[...])
        nfo_ref[...] = jnp.dot(h, w2_ref[...], preferred_element_type=_f32) \
            + b2_ref[...] + nfb
        coo_ref[...] = co_ref[...] + delta

    full = lambda shape: pl.BlockSpec(shape, lambda i: (0, 0))
    return pl.pallas_call(
        body,
        grid=(N // bn,),
        in_specs=[
            pl.BlockSpec((bn, C), lambda i: (i, 0)),
            pl.BlockSpec((bn, 3), lambda i: (i, 0)),
        ] + [pl.BlockSpec((bn, C), lambda i: (i, 0))] * nparts + [
            pl.BlockSpec((bn, 4), lambda i: (i, 0)),
            full((C, C)),
            full((C, C)),
            full((1, C)),
            full((C, C)),
            full((1, C)),
        ],
        out_specs=[
            pl.BlockSpec((bn, C), lambda i: (i, 0)),
            pl.BlockSpec((bn, 3), lambda i: (i, 0)),
        ],
        out_shape=[
            jax.ShapeDtypeStruct((N, C), _f32),
            jax.ShapeDtypeStruct((N, 3), _f32),
        ],
    )(nf, coords, *msg_parts, dsum, wn1a, wn1b, bn1, wn2, bn2)


def kernel(node_feats, coordinates, edge_index, params):
    pe, pn, pa, px = params["phi_e"], params["phi_n"], params["att"], params["phi_x"]
    w1 = pe["W1"]
    w1i = w1[:C]
    w1j = w1[C:2 * C]
    cut = params["bessel_cut_off"]
    amp = jnp.sqrt(2.0 / cut)
    zc16 = jnp.zeros((1, 16), _f32).at[0, :8].set(
        params["z_0k"] / (2.0 * jnp.pi * cut))
    w1g16 = jnp.zeros((16, C), _f32).at[:8].set(w1[2 * C + 1:] * amp)
    w1g16 = w1g16.at[8].set(w1[2 * C]).at[9].set(pe["b1"])
    w_abs = jnp.zeros((1, 16), _f32).at[0, 8].set(1.0)
    b1 = jnp.zeros((1, 16), _f32).at[0, 9].set(1.0)
    b2 = pe["b2"].reshape(1, C)
    wax1 = jnp.concatenate([pa["W1"], px["W1"]], axis=1)
    bax1 = jnp.concatenate([pa["b1"], px["b1"]]).reshape(1, 2 * C)
    wax2 = jnp.zeros((2 * C, C), _f32)
    wax2 = wax2.at[:C, 0].set(pa["W2"][:, 0]).at[C:, 1].set(px["W2"][:, 0])
    bax2 = jnp.zeros((1, C), _f32)
    bax2 = bax2.at[0, 0].set(pa["b2"][0]).at[0, 1].set(px["b2"][0])
    wn1 = pn["W1"]
    wn1a, wn1b = wn1[:C], wn1[C:]
    bn1 = pn["b1"].reshape(1, C)
    bn2 = pn["b2"].reshape(1, C)

    snd = edge_index[0]
    rcv = edge_index[1]
    cxyz = coordinates.T
    zrows = jnp.zeros((NPAD, C), _f32)

    ts, tr = _tables_tc(node_feats, w1i, w1j)
    rv = _sc_geo(cxyz[0], cxyz[1], cxyz[2], snd, rcv).reshape(E, 4)

    msg_parts, dparts_list, att_parts = [], [], []
    off = 0
    for ep in PART_SIZES:
        snd_p = lax.slice(snd, (off,), (off + ep,))
        rcv_p = lax.slice(rcv, (off,), (off + ep,))
        rv_p = lax.slice(rv, (off, 0), (off + ep, 4))
        ga, gb = _sc_gather(ts, tr, snd_p, rcv_p, ep)
        msg, dvec, att = _edge_tc(ga, gb, rv_p, zc16, w1g16, w_abs, b1,
                                  pe["W2"], b2, wax1, bax1, wax2, bax2, ep)
        partm = _sc_scatter_msg(msg, snd_p, zrows, ep)
        partd = _sc_scatter_delta(dvec.reshape(ep * 4), snd_p, ep)
        msg_parts.extend([partm[0], partm[1]])
        dparts_list.append(partd)
        att_parts.append(att)
        off += ep

    dparts2d = jnp.concatenate(dparts_list).reshape(len(PART_SIZES) * NW, N * 4)
    dsum = _delta_combine(dparts2d).reshape(N, 4)
    nf_new, co_new = _node_tc(node_feats, coordinates, msg_parts, dsum,
                              wn1a, wn1b, bn1, pn["W2"], bn2)
    att_full = jnp.concatenate(att_parts, axis=0)
    return nf_new, co_new, att_full
